# baseline (speedup 1.0000x reference)
import jax
import jax.numpy as jnp
from jax.experimental import pallas as pl

_N_CH = 85
_N_ANCH = 3
_NPIX = 1024
_FS = 32
_STRIDE = 32.0
_AW = (116.0 / 32.0, 156.0 / 32.0, 373.0 / 32.0)
_AH = (90.0 / 32.0, 198.0 / 32.0, 326.0 / 32.0)


def _decode_kernel(x_ref, o_ref):
    x = x_ref[0]
    pix = jax.lax.broadcasted_iota(jnp.int32, (_N_CH, _NPIX), 1)
    xs = (pix & (_FS - 1)).astype(jnp.float32)
    ys = (pix >> 5).astype(jnp.float32)
    ch = jax.lax.broadcasted_iota(jnp.int32, (_N_CH, _NPIX), 0)
    for a in range(_N_ANCH):
        xa = x[a * _N_CH:(a + 1) * _N_CH, :]
        sig = jax.nn.sigmoid(xa)
        ex = jnp.exp(xa)
        val = jnp.where(
            ch == 0, (sig + xs) * _STRIDE,
            jnp.where(
                ch == 1, (sig + ys) * _STRIDE,
                jnp.where(
                    ch == 2, ex * _AW[a] * _STRIDE,
                    jnp.where(ch == 3, ex * _AH[a] * _STRIDE, sig))))
        o_ref[0, a * _NPIX:(a + 1) * _NPIX, :] = val.T


@jax.jit
def kernel(xin):
    b = xin.shape[1]
    x = xin[0].reshape(b, _N_ANCH * _N_CH, _NPIX)
    return pl.pallas_call(
        _decode_kernel,
        grid=(b,),
        in_specs=[pl.BlockSpec((1, _N_ANCH * _N_CH, _NPIX), lambda i: (i, 0, 0))],
        out_specs=pl.BlockSpec((1, _N_ANCH * _NPIX, _N_CH), lambda i: (i, 0, 0)),
        out_shape=jax.ShapeDtypeStruct((b, _N_ANCH * _NPIX, _N_CH), jnp.float32),
    )(x)
